# Initial kernel scaffold; baseline (speedup 1.0000x reference)
import jax
import jax.numpy as jnp
from jax.experimental import pallas as pl


def kernel(state, action, task_indicator, w_cx1_1, b_cx1_1, w_cx1_2, b_cx1_2, w_cx2_1, b_cx2_1, w_cx2_2, b_cx2_2, w_cx3_1, b_cx3_1, w_cx3_2, b_cx3_2, w1, b1, w2, b2, w3, b3, w4, b4):
    raise NotImplementedError("write your pallas kernel here")


# speedup vs baseline: 8.4456x; 8.4456x over previous
import jax
import jax.numpy as jnp
from jax.experimental import pallas as pl
from jax.experimental.pallas import tpu as pltpu

_ROWS = 256


def _row_argmax(z):
    n = z.shape[1]
    zmax = jnp.max(z, axis=1, keepdims=True)
    ii = jax.lax.broadcasted_iota(jnp.int32, z.shape, 1)
    return jnp.min(jnp.where(z >= zmax, ii, n), axis=1, keepdims=True)


def _kwta(x, k):
    b = jax.lax.bitcast_convert_type(x, jnp.int32)
    m = b ^ (jax.lax.shift_right_arithmetic(b, 31) & jnp.int32(0x7FFFFFFF))
    cnt = jnp.sum((m >= 0).astype(jnp.int32), axis=1, keepdims=True)
    thr = jnp.where(cnt >= k, jnp.int32(0), jnp.int32(-(2**31)))
    for bit in range(30, -1, -1):
        trial = thr | jnp.int32(1 << bit)
        cnt = jnp.sum((m >= trial).astype(jnp.int32), axis=1, keepdims=True)
        thr = jnp.where(cnt >= k, trial, thr)
    return jnp.where(m >= thr, x, x * jnp.float32(1.0 / 3.0))


def _body(ci_ref, wc11_ref, bc11_ref, wc12_ref, bc12_ref,
          wc21_ref, bc21_ref, wc22_ref, bc22_ref,
          wc31_ref, bc31_ref, wc32_ref, bc32_ref,
          w1_ref, b1_ref, w2_ref, b2_ref, w3_ref, b3_ref,
          w4_ref, b4_ref, out_ref):
    def dot(a, b):
        return jax.lax.dot_general(a, b, (((1,), (0,)), ((), ())),
                                   preferred_element_type=jnp.float32)

    ci = ci_ref[...]
    k1 = _row_argmax(dot(jnp.tanh(dot(ci, wc11_ref[...]) + bc11_ref[...]),
                         wc12_ref[...]) + bc12_ref[...])
    k2 = _row_argmax(dot(jnp.tanh(dot(ci, wc21_ref[...]) + bc21_ref[...]),
                         wc22_ref[...]) + bc22_ref[...])
    k3 = _row_argmax(dot(jnp.tanh(dot(ci, wc31_ref[...]) + bc31_ref[...]),
                         wc32_ref[...]) + bc32_ref[...])
    x = _kwta(dot(ci, w1_ref[...]) + b1_ref[...], k1)
    x = _kwta(dot(x, w2_ref[...]) + b2_ref[...], k2)
    x = _kwta(dot(x, w3_ref[...]) + b3_ref[...], k3)
    out_ref[...] = dot(x, w4_ref[...]) + b4_ref[...]


def kernel(state, action, task_indicator,
           w_cx1_1, b_cx1_1, w_cx1_2, b_cx1_2,
           w_cx2_1, b_cx2_1, w_cx2_2, b_cx2_2,
           w_cx3_1, b_cx3_1, w_cx3_2, b_cx3_2,
           w1, b1, w2, b2, w3, b3, w4, b4):
    b = state.shape[0]
    rows = min(_ROWS, b)
    ci = jnp.concatenate([state, task_indicator, action], axis=1)
    inp = ci.shape[1]
    h = w4.shape[1]

    def wspec(w):
        return pl.BlockSpec(w.shape, lambda i: (0, 0))

    weights = [w_cx1_1, b_cx1_1.reshape(1, -1), w_cx1_2, b_cx1_2.reshape(1, -1),
               w_cx2_1, b_cx2_1.reshape(1, -1), w_cx2_2, b_cx2_2.reshape(1, -1),
               w_cx3_1, b_cx3_1.reshape(1, -1), w_cx3_2, b_cx3_2.reshape(1, -1),
               w1, b1.reshape(1, -1), w2, b2.reshape(1, -1),
               w3, b3.reshape(1, -1), w4, b4.reshape(1, -1)]

    return pl.pallas_call(
        _body,
        grid=(b // rows,),
        in_specs=[pl.BlockSpec((rows, inp), lambda i: (i, 0))] +
                 [wspec(w) for w in weights],
        out_specs=pl.BlockSpec((rows, h), lambda i: (i, 0)),
        out_shape=jax.ShapeDtypeStruct((b, h), jnp.float32),
    )(ci, *weights)
